# baseline (speedup 1.0000x reference)
import functools

import jax
import jax.numpy as jnp
from jax import lax
from jax.experimental import pallas as pl
from jax.experimental.pallas import tpu as pltpu
from jax.experimental.pallas import tpu_sc as plsc

BC = 2048
NEG_INF = float("-inf")
IDX_PAD = 2**31 - 1


_LANES = 16
_NW = 32
_K32 = 32


def _permute(x, perm):
    dn = lax.GatherDimensionNumbers(
        offset_dims=(), collapsed_slice_dims=(0,), start_index_map=(0,))
    return lax.gather(x, perm[:, None], dn, (1,),
                      mode=lax.GatherScatterMode.PROMISE_IN_BOUNDS)


def _lex_gt(av, ai, bv, bi):
    return (av > bv) | ((av == bv) & (ai < bi))


def _cmp_ex(v, i, j, k):
    lane = lax.iota(jnp.int32, _LANES)
    perm = lax.bitwise_xor(lane, j)
    pv = _permute(v, perm)
    pi = _permute(i, perm)
    win = _lex_gt(v, i, pv, pi)
    keep_hi = ((lane & k) == 0) == ((lane & j) == 0)
    sel = keep_hi == win
    return jnp.where(sel, v, pv), jnp.where(sel, i, pi)


def _bitonic_sort16(v, i):
    for k in (2, 4, 8, 16):
        j = k // 2 if k < 16 else 8
        jj = j
        while jj >= 1:
            v, i = _cmp_ex(v, i, jj, k)
            jj //= 2
    return v, i


def _bitonic_merge16(v, i):
    for jj in (8, 4, 2, 1):
        v, i = _cmp_ex(v, i, jj, 16)
    return v, i


def _merge_chunk(cv, ci, v0, i0, v1, i1):
    cv, ci = _bitonic_sort16(cv, ci)
    rv = lax.rev(cv, (0,))
    ri = lax.rev(ci, (0,))
    w = _lex_gt(v1, i1, rv, ri)
    mv = jnp.where(w, v1, rv)
    mi = jnp.where(w, i1, ri)
    mv, mi = _bitonic_merge16(mv, mi)
    rv2 = lax.rev(mv, (0,))
    ri2 = lax.rev(mi, (0,))
    w2 = _lex_gt(v0, i0, rv2, ri2)
    hv = jnp.where(w2, v0, rv2)
    hi = jnp.where(w2, i0, ri2)
    lv = jnp.where(w2, rv2, v0)
    li = jnp.where(w2, ri2, i0)
    v0, i0 = _bitonic_merge16(hv, hi)
    v1, i1 = _bitonic_merge16(lv, li)
    return v0, i0, v1, i1


def _scan_topk(load_chunk, n_chunks, init_thr_v, init_thr_i):
    z0 = jnp.full((_LANES,), init_thr_v, jnp.float32)
    ii = jnp.full((_LANES,), init_thr_i, jnp.int32)

    def body(kk, carry):
        v0, i0, v1, i1 = carry
        cv, ci = load_chunk(kk)
        thr = jnp.min(v1)
        hit = jnp.sum((cv >= thr).astype(jnp.int32)) > 0

        def do(args):
            return _merge_chunk(cv, ci, *args)

        def skip(args):
            return args

        return lax.cond(hit, do, skip, (v0, i0, v1, i1))

    return lax.fori_loop(0, n_chunks, body, (z0, ii, z0, ii))


def _sc_topk(keys, cand):
    b, _, npad = keys.shape
    _, n, dmod = cand.shape
    slc = npad // _NW
    n_chunks = slc // _LANES
    n_jobs = b * 3
    mesh = plsc.VectorSubcoreMesh(core_axis_name="c", subcore_axis_name="s")

    @functools.partial(
        pl.kernel,
        mesh=mesh,
        out_type=[
            jax.ShapeDtypeStruct((b, 64), jnp.int32),
            jax.ShapeDtypeStruct((b, 64, dmod), jnp.float32),
        ],
        scratch_types=[
            pltpu.VMEM((slc,), jnp.float32),
            pltpu.VMEM((_K32,), jnp.float32),
            pltpu.VMEM((_K32,), jnp.int32),
            pltpu.VMEM((_NW, _K32), jnp.float32),
            pltpu.VMEM((_NW, _K32), jnp.int32),
            pltpu.VMEM((_NW, _LANES), jnp.float32),
            pltpu.VMEM((64,), jnp.int32),
            pltpu.VMEM((64, 64), jnp.float32),
            pltpu.VMEM_SHARED((_NW, _LANES), jnp.float32),
            pltpu.VMEM_SHARED((_NW, _K32), jnp.float32),
            pltpu.VMEM_SHARED((_NW, _K32), jnp.int32),
            pltpu.VMEM_SHARED((b, 64), jnp.int32),
            pltpu.SemaphoreType.DMA,
        ],
    )
    def k(keys_hbm, cand_hbm, idx_out, emb_out, kv, wv, wi, mv, mi,
          mxv, idxv, embv, sp_max, sp_v, sp_i, sp_sel, sem):
        wid = lax.axis_index("s") * 2 + lax.axis_index("c")
        base = wid * slc

        def job(jj, _):
            bb = jj // 3
            rr = jj % 3
            pltpu.sync_copy(keys_hbm.at[bb, rr, pl.ds(base, slc)], kv)

            def mx_body(kk, acc):
                return jnp.maximum(acc, kv[pl.ds(kk * _LANES, _LANES)])

            tmax = lax.fori_loop(
                0, n_chunks, mx_body,
                jnp.full((_LANES,), NEG_INF, jnp.float32))
            wv[pl.ds(0, _LANES)] = tmax
            pltpu.sync_copy(wv.at[pl.ds(0, _LANES)], sp_max.at[wid])
            plsc.subcore_barrier()
            pltpu.sync_copy(sp_max, mxv)

            def tau_body(tt, acc):
                return jnp.minimum(
                    acc, jnp.max(mxv[tt, pl.ds(0, _LANES)]))

            tau0 = lax.fori_loop(0, _NW, tau_body, jnp.float32(jnp.inf))

            def load_local(kk):
                cv = kv[pl.ds(kk * _LANES, _LANES)]
                ci = (base + kk * _LANES) + lax.iota(jnp.int32, _LANES)
                return cv, ci

            v0, i0, v1, i1 = _scan_topk(load_local, n_chunks, tau0, IDX_PAD)
            wv[pl.ds(0, _LANES)] = v0
            wv[pl.ds(_LANES, _LANES)] = v1
            wi[pl.ds(0, _LANES)] = i0
            wi[pl.ds(_LANES, _LANES)] = i1
            pltpu.sync_copy(wv, sp_v.at[wid])
            pltpu.sync_copy(wi, sp_i.at[wid])
            plsc.subcore_barrier()

            @pl.when(wid == jj)
            def _():
                pltpu.sync_copy(sp_v, mv)
                pltpu.sync_copy(sp_i, mi)

                def load_m(kk):
                    t = kk // 2
                    off = (kk % 2) * _LANES
                    return (mv[t, pl.ds(off, _LANES)],
                            mi[t, pl.ds(off, _LANES)])

                f0, g0, f1, g1 = _scan_topk(
                    load_m, 2 * _NW, NEG_INF, IDX_PAD)
                wi[pl.ds(0, _LANES)] = g0
                wi[pl.ds(_LANES, _LANES)] = g1

                @pl.when(rr == 0)
                def _():
                    pltpu.sync_copy(wi, sp_sel.at[bb, pl.ds(0, _K32)])

                @pl.when(rr == 1)
                def _():
                    pltpu.sync_copy(wi.at[pl.ds(0, _LANES)],
                                    sp_sel.at[bb, pl.ds(32, _LANES)])

                @pl.when(rr == 2)
                def _():
                    pltpu.sync_copy(wi.at[pl.ds(0, _LANES)],
                                    sp_sel.at[bb, pl.ds(48, _LANES)])

            plsc.subcore_barrier()
            return 0

        lax.fori_loop(0, n_jobs, job, 0)

        @pl.when(wid < b)
        def _():
            pltpu.sync_copy(sp_sel.at[wid], idxv)
            pltpu.async_copy(cand_hbm.at[wid].at[idxv], embv, sem).wait()
            pltpu.sync_copy(embv, emb_out.at[wid])
            pltpu.sync_copy(idxv, idx_out.at[wid])

    return k(keys, cand)


def _refsum(x):
    acc = x[0:8] + x[8:16]
    for a in range(2, 8):
        acc = acc + x[8 * a:8 * a + 8]
    acc = acc[0:4] + acc[4:8]
    acc = acc[0:2] + acc[2:4]
    return acc[0:1] + acc[1:2]


def _keys_body(q_ref, m_ref, t_ref, c_ref, out_ref, *, n_valid):
    j = pl.program_id(1)
    qt = jnp.transpose(q_ref[0])
    mt = jnp.transpose(m_ref[0])
    ct = jnp.transpose(c_ref[0])

    qwt = qt * mt
    diff = ct - qt
    wdiff = diff * mt
    sd = _refsum(diff * wdiff)
    s1 = _refsum(qwt * ct)
    s3 = _refsum(ct * ct)
    s2 = _refsum((ct * mt) * ct)

    qn = jnp.sqrt(_refsum(qt * qt))
    qmq = _refsum((qt * mt) * qt)

    dist = jnp.sqrt(jnp.clip(sd, 1e-8, None))
    k_near = -dist
    ch0 = s1 / (qn * jnp.clip(jnp.sqrt(s3), 1e-8, None) + 1e-8)
    prod = qmq * s2
    wedge = jnp.sqrt(jnp.clip(prod - s1 * s1, 0.0, None) + 1e-8)
    ch1 = wedge / jnp.clip(jnp.sqrt(prod), 1e-8, None)

    col = j * BC + lax.broadcasted_iota(jnp.int32, (1, BC), 1)
    valid = col < n_valid
    ninf = jnp.full((1, BC), NEG_INF, jnp.float32)
    ch0 = jnp.where(valid, ch0, ninf)
    ch1 = jnp.where(valid, ch1, ninf)
    k_near = jnp.where(valid, k_near, ninf)

    out_ref[0] = jnp.concatenate(
        [ch0, ch1, k_near, jnp.zeros((1, BC), jnp.float32)], axis=0)


def _compute_keys(q, c, m, t):
    b, n, dmod = c.shape
    nb = -(-n // BC)
    npad = nb * BC
    body = functools.partial(_keys_body, n_valid=n)
    return pl.pallas_call(
        body,
        grid=(b, nb),
        in_specs=[
            pl.BlockSpec((1, 1, dmod), lambda i, j: (i, 0, 0)),
            pl.BlockSpec((1, 1, dmod), lambda i, j: (i, 0, 0)),
            pl.BlockSpec((1, 1, dmod), lambda i, j: (i, 0, 0)),
            pl.BlockSpec((1, BC, dmod), lambda i, j: (i, j, 0)),
        ],
        out_specs=pl.BlockSpec((1, 4, BC), lambda i, j: (i, 0, j)),
        out_shape=jax.ShapeDtypeStruct((b, 4, npad), jnp.float32),
    )(q[:, None, :], m[:, None, :], t[:, None, :], c)


def _rescore_body(q_ref, m_ref, t_ref, e_ref, out_ref):
    q = q_ref[0]
    m = m_ref[0]
    t = t_ref[0]
    e = e_ref[0]

    diff = e - q
    wdiff = diff * m
    dsq = jnp.sum(diff * wdiff, axis=-1, keepdims=True)
    dist = jnp.sqrt(jnp.clip(dsq, 1e-8, None))
    qw = q * m
    dot_s = jnp.sum(qw * e, axis=-1, keepdims=True)
    qn = jnp.sqrt(jnp.sum(q * q))
    cn = jnp.sqrt(jnp.sum(e * e, axis=-1, keepdims=True))
    ch0 = dot_s / (qn * jnp.clip(cn, 1e-8, None) + 1e-8)
    qmq = jnp.sum(qw * q)
    cm = jnp.sum((e * m) * e, axis=-1, keepdims=True)
    prod = qmq * cm
    wedge = jnp.sqrt(jnp.clip(prod - dot_s * dot_s, 0.0, None) + 1e-8)
    ch1 = wedge / jnp.clip(jnp.sqrt(prod), 1e-8, None)
    ch2 = jnp.mean(jnp.abs(qw * e), axis=-1, keepdims=True)
    ch3 = jnp.sum(jnp.abs(diff * t), axis=-1, keepdims=True) / (dist + 0.001)
    ch4 = -1.0 / (dist * dist + 0.001)
    rank = lax.broadcasted_iota(jnp.int32, (64, 1), 0).astype(
        jnp.float32) / 63.0
    out_ref[0] = jnp.concatenate([ch0, ch1, ch2, ch3, ch4, rank], axis=1)


def _rescore(q, m, t, emb):
    b, k, dmod = emb.shape
    return pl.pallas_call(
        _rescore_body,
        grid=(b,),
        in_specs=[
            pl.BlockSpec((1, 1, dmod), lambda i: (i, 0, 0)),
            pl.BlockSpec((1, 1, dmod), lambda i: (i, 0, 0)),
            pl.BlockSpec((1, 1, dmod), lambda i: (i, 0, 0)),
            pl.BlockSpec((1, k, dmod), lambda i: (i, 0, 0)),
        ],
        out_specs=pl.BlockSpec((1, k, 6), lambda i: (i, 0, 0)),
        out_shape=jax.ShapeDtypeStruct((b, k, 6), jnp.float32),
    )(q[:, None, :], m[:, None, :], t[:, None, :], emb)


def kernel(query_embedding, candidate_embeddings, metric, transport):
    b, n, dmod = candidate_embeddings.shape
    keys = _compute_keys(query_embedding, candidate_embeddings, metric,
                         transport)
    _, ni = lax.top_k(keys[:, 2, :n], 32)
    _, ai = lax.top_k(keys[:, 0, :n], 16)
    _, ri = lax.top_k(keys[:, 1, :n], 16)
    sel = jnp.concatenate([ni, ai, ri], axis=1)
    emb = jnp.take_along_axis(candidate_embeddings, sel[:, :, None], axis=1)
    scores = _rescore(query_embedding, metric, transport, emb)
    return emb, scores, sel
